# baseline (speedup 1.0000x reference)
import jax
import jax.numpy as jnp
from jax import lax
from jax.experimental import pallas as pl
from jax.experimental.pallas import tpu as pltpu
from jax.experimental.pallas import tpu_sc as plsc

D_MODEL = 64
SCALE = 8.0
NC = 2
NS = 16
NW = NC * NS
CHUNK = 128
LANES = 16


def _emb_body(table_hbm, x_hbm, out_hbm, idx_v, rows_v, gsem):
    b_per_w = x_hbm.shape[0] // NW
    wid = lax.axis_index("s") * NC + lax.axis_index("c")
    base = wid * b_per_w
    pltpu.sync_copy(x_hbm.at[pl.ds(base, b_per_w)], idx_v)

    @pl.loop(0, b_per_w // CHUNK)
    def _chunk(c):
        off = c * CHUNK
        pltpu.async_copy(
            table_hbm.at[idx_v.at[pl.ds(off, CHUNK)]], rows_v, gsem
        ).wait()

        @pl.loop(0, CHUNK)
        def _row(r):
            @pl.loop(0, D_MODEL, step=LANES)
            def _col(j):
                sl = (r, pl.ds(j, LANES))
                rows_v[sl] = rows_v[sl] * SCALE

        pltpu.sync_copy(rows_v, out_hbm.at[pl.ds(base + off, CHUNK)])


def kernel(x, table):
    B = x.shape[0] * x.shape[1]
    xf = x.reshape(B)
    b_per_w = B // NW
    mesh = plsc.VectorSubcoreMesh(core_axis_name="c", subcore_axis_name="s")
    run = pl.kernel(
        _emb_body,
        out_type=jax.ShapeDtypeStruct((B, D_MODEL), jnp.float32),
        mesh=mesh,
        compiler_params=pltpu.CompilerParams(use_tc_tiling_on_sc=False),
        scratch_types=[
            pltpu.VMEM((b_per_w,), jnp.int32),
            pltpu.VMEM((CHUNK, D_MODEL), jnp.float32),
            pltpu.SemaphoreType.DMA,
        ],
    )
    out = run(table, xf)
    return out.reshape(x.shape[0], x.shape[1], D_MODEL)


# speedup vs baseline: 1.2097x; 1.2097x over previous
import jax
import jax.numpy as jnp
from jax import lax
from jax.experimental import pallas as pl
from jax.experimental.pallas import tpu as pltpu
from jax.experimental.pallas import tpu_sc as plsc

D_MODEL = 64
SCALE = 8.0
NC = 2
NS = 16
NW = NC * NS
CHUNK = 128
LANES = 16
NBUF = 4
ROW_UNROLL = 4


def _emb_body(table_hbm, x_hbm, out_hbm, idx_v, rows_in, rows_out, gsem, osem):
    b_per_w = x_hbm.shape[0] // NW
    nchunks = b_per_w // CHUNK
    ngroups = nchunks // NBUF
    wid = lax.axis_index("s") * NC + lax.axis_index("c")
    base = wid * b_per_w
    pltpu.sync_copy(x_hbm.at[pl.ds(base, b_per_w)], idx_v)

    def gather_start(b, off):
        pltpu.make_async_copy(
            table_hbm.at[idx_v.at[pl.ds(off, CHUNK)]],
            rows_in.at[b], gsem.at[b],
        ).start()

    def gather_wait(b):
        pltpu.make_async_copy(
            table_hbm.at[idx_v.at[pl.ds(0, CHUNK)]],
            rows_in.at[b], gsem.at[b],
        ).wait()

    def out_start(b, off):
        pltpu.make_async_copy(
            rows_out.at[b], out_hbm.at[pl.ds(base + off, CHUNK)], osem.at[b],
        ).start()

    def out_wait(b):
        pltpu.make_async_copy(
            rows_out.at[b], out_hbm.at[pl.ds(base, CHUNK)], osem.at[b],
        ).wait()

    def scale(b):
        @pl.loop(0, CHUNK, step=ROW_UNROLL)
        def _rows(r0):
            for dr in range(ROW_UNROLL):
                for j in range(D_MODEL // LANES):
                    sl = (r0 + dr, pl.ds(j * LANES, LANES))
                    rows_out.at[b][sl] = rows_in.at[b][sl] * SCALE

    for b in range(NBUF):
        gather_start(b, b * CHUNK)

    for b in range(NBUF):
        gather_wait(b)
        scale(b)
        gather_start(b, (NBUF + b) * CHUNK)
        out_start(b, b * CHUNK)

    @pl.loop(1, ngroups)
    def _group(g):
        for b in range(NBUF):
            off = (g * NBUF + b) * CHUNK
            gather_wait(b)
            out_wait(b)
            scale(b)

            @pl.when(g < ngroups - 1)
            def _():
                gather_start(b, off + NBUF * CHUNK)

            out_start(b, off)

    for b in range(NBUF):
        out_wait(b)


def kernel(x, table):
    B = x.shape[0] * x.shape[1]
    xf = x.reshape(B)
    b_per_w = B // NW
    mesh = plsc.VectorSubcoreMesh(core_axis_name="c", subcore_axis_name="s")
    run = pl.kernel(
        _emb_body,
        out_type=jax.ShapeDtypeStruct((B, D_MODEL), jnp.float32),
        mesh=mesh,
        compiler_params=pltpu.CompilerParams(use_tc_tiling_on_sc=False),
        scratch_types=[
            pltpu.VMEM((b_per_w,), jnp.int32),
            pltpu.VMEM((NBUF, CHUNK, D_MODEL), jnp.float32),
            pltpu.VMEM((NBUF, CHUNK, D_MODEL), jnp.float32),
            pltpu.SemaphoreType.DMA((NBUF,)),
            pltpu.SemaphoreType.DMA((NBUF,)),
        ],
    )
    out = run(table, xf)
    return out.reshape(x.shape[0], x.shape[1], D_MODEL)
